# Initial kernel scaffold; baseline (speedup 1.0000x reference)
import jax
import jax.numpy as jnp
from jax.experimental import pallas as pl


def kernel(x, conv1_w, conv1_b, conv2_w, conv2_b, conv3_w, conv3_b, fc1_w0, fc1_w1, fc1_w2, fc1_b0, fc1_b1, fc1_b2, fc2_w0, fc2_w1, fc2_w2, fc2_b0, fc2_b1, fc2_b2, bb_e1, bb_e2, bb_e3, bb_map):
    raise NotImplementedError("write your pallas kernel here")


# speedup vs baseline: 3.2016x; 3.2016x over previous
import functools

import jax
import jax.numpy as jnp
from jax.experimental import pallas as pl
from jax.experimental.pallas import tpu as pltpu


def _pool3_rows(c, scr_ref):
    n = c.shape[0] // 3
    scr_ref[...] = c
    return jnp.maximum(
        jnp.maximum(scr_ref[pl.ds(0, n, stride=3)],
                    scr_ref[pl.ds(1, n, stride=3)]),
        scr_ref[pl.ds(2, n, stride=3)])


def _conv_stack_kernel(r1_ref, w1_ref, b1_ref, w2_ref, b2_ref,
                       w3_ref, b3_ref, o_ref, y1_ref, s1_ref, s2_ref):
    w1 = w1_ref[...]
    b1 = b1_ref[...]
    for r in range(19):
        band = []
        for k in range(3):
            cr = 3 * r + k
            band.append(jnp.concatenate(
                [r1_ref[cr + dy] for dy in range(10)], axis=1))
        a = jnp.concatenate(band, axis=0)
        c = jnp.dot(a, w1, preferred_element_type=jnp.float32)
        c = jnp.maximum(jnp.maximum(c[0:57], c[57:114]), c[114:171])
        c = jnp.maximum(c + b1, 0.0)
        y1_ref[r] = _pool3_rows(c, s1_ref)

    for g in range(3):
        w2 = w2_ref[g]
        b2 = b2_ref[g]
        r2 = jnp.concatenate(
            [y1_ref[:, dx:dx + 15, 32 * g:32 * (g + 1)]
             for dx in range(5)], axis=2)
        y2rows = []
        for r in range(5):
            band = []
            for k in range(3):
                cr = 3 * r + k
                band.append(jnp.concatenate(
                    [r2[cr + dy] for dy in range(5)], axis=1))
            a = jnp.concatenate(band, axis=0)
            c = jnp.dot(a, w2, preferred_element_type=jnp.float32)
            c = jnp.maximum(jnp.maximum(c[0:15], c[15:30]), c[30:45])
            c = jnp.maximum(c + b2, 0.0)
            y2rows.append(_pool3_rows(c, s2_ref)[None])
        y2 = jnp.concatenate(y2rows, axis=0)

        r3 = jnp.concatenate(
            [y2[:, dx:dx + 3, :] for dx in range(3)], axis=2)
        band = [jnp.concatenate([r3[cr + dy] for dy in range(3)], axis=1)
                for cr in range(3)]
        a = jnp.concatenate(band, axis=0)
        c = jnp.dot(a, w3_ref[g], preferred_element_type=jnp.float32)
        c = jnp.maximum(jnp.maximum(c[0:3], c[3:6]), c[6:9])
        c = jnp.maximum(c + b3_ref[g], 0.0)
        o_ref[pl.ds(g, 1), :] = jnp.max(c, axis=0, keepdims=True)


def _head_kernel(f_ref,
                 w1a_ref, b1a_ref, w2a_ref, b2a_ref,
                 w1b_ref, b1b_ref, w2b_ref, b2b_ref,
                 w1c_ref, b1c_ref, w2c_ref, b2c_ref,
                 e1_ref, e2_ref, e3_ref, mp_ref, o_ref):
    def probs(g, w1_ref, b1_ref, w2_ref, b2_ref):
        h = jnp.dot(f_ref[g], w1_ref[...], preferred_element_type=jnp.float32)
        h = jnp.maximum(h + b1_ref[...], 0.0)
        z = jnp.dot(h, w2_ref[...], preferred_element_type=jnp.float32)
        z = z + b2_ref[...]
        e = jnp.exp(z - jnp.max(z, axis=-1, keepdims=True))
        return e / jnp.sum(e, axis=-1, keepdims=True)

    p1 = probs(0, w1a_ref, b1a_ref, w2a_ref, b2a_ref)
    p2 = probs(1, w1b_ref, b1b_ref, w2b_ref, b2b_ref)
    p3 = probs(2, w1c_ref, b1c_ref, w2c_ref, b2c_ref)
    q = (jnp.dot(p1, e1_ref[...], preferred_element_type=jnp.float32)
         * jnp.dot(p2, e2_ref[...], preferred_element_type=jnp.float32)
         * jnp.dot(p3, e3_ref[...], preferred_element_type=jnp.float32))
    o_ref[...] = jnp.dot(q, mp_ref[...], preferred_element_type=jnp.float32)


def kernel(x, conv1_w, conv1_b, conv2_w, conv2_b, conv3_w, conv3_b,
           fc1_w0, fc1_w1, fc1_w2, fc1_b0, fc1_b1, fc1_b2,
           fc2_w0, fc2_w1, fc2_w2, fc2_b0, fc2_b1, fc2_b2,
           bb_e1, bb_e2, bb_e3, bb_map):
    B = x.shape[0]
    xh = jnp.transpose(x, (0, 2, 3, 1)).astype(jnp.float32)
    r1 = jnp.concatenate([xh[:, :, dx:dx + 57, :] for dx in range(10)],
                         axis=3)
    w1 = conv1_w.reshape(3, 10, 10, 96).transpose(1, 2, 0, 3).reshape(300, 96)
    w2 = conv2_w.reshape(3, 32, 5, 5, 64).transpose(0, 2, 3, 1, 4)
    w2 = w2.reshape(3, 800, 64)
    w3 = conv3_w.reshape(3, 64, 3, 3, 128).transpose(0, 2, 3, 1, 4)
    w3 = w3.reshape(3, 576, 128)

    feats = pl.pallas_call(
        _conv_stack_kernel,
        out_shape=jax.ShapeDtypeStruct((B, 3, 128), jnp.float32),
        grid=(B,),
        in_specs=[
            pl.BlockSpec((None, 66, 57, 30), lambda b: (b, 0, 0, 0)),
            pl.BlockSpec((300, 96), lambda b: (0, 0)),
            pl.BlockSpec((1, 96), lambda b: (0, 0)),
            pl.BlockSpec((3, 800, 64), lambda b: (0, 0, 0)),
            pl.BlockSpec((3, 1, 64), lambda b: (0, 0, 0)),
            pl.BlockSpec((3, 576, 128), lambda b: (0, 0, 0)),
            pl.BlockSpec((3, 1, 128), lambda b: (0, 0, 0)),
        ],
        out_specs=pl.BlockSpec((None, 3, 128), lambda b: (b, 0, 0)),
        scratch_shapes=[pltpu.VMEM((19, 19, 96), jnp.float32),
                        pltpu.VMEM((57, 96), jnp.float32),
                        pltpu.VMEM((15, 64), jnp.float32)],
        compiler_params=pltpu.CompilerParams(
            dimension_semantics=("parallel",)),
    )(r1, w1, conv1_b, w2, conv2_b, w3, conv3_b)

    feats = jnp.transpose(feats, (1, 0, 2))
    L = bb_map.shape[1]
    hb = B // 2
    args = (feats,
            fc1_w0, fc1_b0, fc2_w0, fc2_b0,
            fc1_w1, fc1_b1, fc2_w1, fc2_b1,
            fc1_w2, fc1_b2, fc2_w2, fc2_b2,
            bb_e1, bb_e2, bb_e3, bb_map)
    return pl.pallas_call(
        _head_kernel,
        out_shape=jax.ShapeDtypeStruct((B, L), jnp.float32),
        grid=(2,),
        in_specs=[pl.BlockSpec((3, hb, 128), lambda i: (0, i, 0))] + [
            pl.BlockSpec(a.shape, lambda i, nd=a.ndim: (0,) * nd)
            for a in args[1:]],
        out_specs=pl.BlockSpec((hb, L), lambda i: (i, 0)),
        compiler_params=pltpu.CompilerParams(
            dimension_semantics=("parallel",)),
    )(*args)
